# baseline (speedup 1.0000x reference)
import functools

import jax
import jax.numpy as jnp
from jax import lax
from jax.experimental import pallas as pl
from jax.experimental.pallas import tpu as pltpu
from jax.experimental.pallas import tpu_sc as plsc

_F32 = jnp.float32
_I32 = jnp.int32


def _vsqrt(x):
    xi = lax.bitcast_convert_type(x, _I32)
    yi = (xi >> 1) + jnp.int32(0x1FBD1DF5)
    y = lax.bitcast_convert_type(yi, _F32)
    for _ in range(4):
        y = 0.5 * (y + x / y)
    return y


def _splat_i32(v):
    return jnp.zeros((16,), _I32) + v


def _sc_body(fd_hbm, out_hbm, fdv, mt, d0, d1, d2, d3, pp, hs, mcol,
             outv, shm, shp, shd, sem):
    cid = lax.axis_index("c")
    sid = lax.axis_index("s")
    lanes = lax.iota(_I32, 16)
    zeros = jnp.zeros((16,), _F32)

    def hsum_full(x):
        for s in (8, 4, 2, 1):
            hs[pl.ds(0, 16)] = x
            x = x + plsc.load_gather(hs, [lanes ^ s])
        return x

    @pl.when((cid == 0) & (sid < 8))
    def _():
        p = sid
        dirx = p < 4
        tb = p % 4
        t = 1 + tb // 2
        b = tb % 2
        base1 = b * 1024
        base2 = (t * 2 + b) * 1024
        pltpu.sync_copy(fd_hbm.at[pl.ds(base1, 1024)],
                        fdv.at[pl.ds(base1, 1024)])
        pltpu.sync_copy(fd_hbm.at[pl.ds(base2, 1024)],
                        fdv.at[pl.ds(base2, 1024)])
        rowmul = jnp.where(dirx, 128, 16)
        wmul = jnp.where(dirx, 16, 128)
        scale = jnp.where(dirx, _F32(0.125), _F32(1.0))

        for g in range(4):
            for k in range(16):
                cell = g * 16 + k
                i = cell // 8
                j = cell % 8
                offa = base1 + i * rowmul
                offb = base2 + j * rowmul
                acc = zeros
                for w in range(8):
                    av = fdv[pl.ds(offa + w * wmul, 16)]
                    bv = fdv[pl.ds(offb + w * wmul, 16)]
                    dv = av - bv
                    acc = acc + dv * dv
                hs[pl.ds(k * 16, 16)] = acc
            totals = zeros
            for ch in range(16):
                totals = totals + plsc.load_gather(hs, [lanes * 16 + ch])
            mvec = _vsqrt(totals * scale) + _F32(1e-08)
            mcol[pl.ds(g * 16, 16)] = mvec

        pltpu.sync_copy(mcol.at[pl.ds(0, 64)], shm.at[pl.ds(p * 64, 64)])

    plsc.subcore_barrier()

    @pl.when((cid == 0) & (sid == 0))
    def _():
        for r in range(32):
            mcol[pl.ds(512 + r * 16, 16)] = zeros
        pltpu.sync_copy(shm, mcol.at[pl.ds(0, 512)])

        for c in range(64):
            mt[pl.ds(c * 16, 16)] = plsc.load_gather(mcol, [lanes * 64 + c])

        d0[pl.ds(0, 16)] = mt[pl.ds(0, 16)]
        d1[pl.ds(0, 16)] = zeros - 1.0
        d2[pl.ds(0, 16)] = zeros - 1.0
        d3[pl.ds(0, 16)] = zeros + 1.0

        for i in range(1, 8):
            c = i * 8
            d0[pl.ds(c * 16, 16)] = mt[pl.ds(c * 16, 16)] + d0[pl.ds((c - 8) * 16, 16)]
            d1[pl.ds(c * 16, 16)] = zeros + _F32(i - 1)
            d2[pl.ds(c * 16, 16)] = zeros
            d3[pl.ds(c * 16, 16)] = d3[pl.ds((c - 8) * 16, 16)] + 1.0

        for j in range(1, 8):
            d0[pl.ds(j * 16, 16)] = mt[pl.ds(j * 16, 16)] + d0[pl.ds((j - 1) * 16, 16)]
            d1[pl.ds(j * 16, 16)] = zeros
            d2[pl.ds(j * 16, 16)] = zeros + _F32(j - 1)
            d3[pl.ds(j * 16, 16)] = d3[pl.ds((j - 1) * 16, 16)] + 1.0

        for i in range(1, 8):
            for j in range(1, 8):
                c = i * 8 + j
                d0u = d0[pl.ds((c - 8) * 16, 16)]
                d0l = d0[pl.ds((c - 1) * 16, 16)]
                d0g = d0[pl.ds((c - 9) * 16, 16)]
                d3u = d3[pl.ds((c - 8) * 16, 16)]
                d3l = d3[pl.ds((c - 1) * 16, 16)]
                d3g = d3[pl.ds((c - 9) * 16, 16)]
                c1 = d0u / d3u
                c2 = d0l / d3l
                c3 = d0g / d3g
                b1 = (c1 <= c2) & (c1 <= c3)
                b2 = c2 <= c3
                mrow = mt[pl.ds(c * 16, 16)]
                d0[pl.ds(c * 16, 16)] = mrow + jnp.where(b1, d0u, jnp.where(b2, d0l, d0g))
                d1[pl.ds(c * 16, 16)] = jnp.where((~b1) & b2, _F32(i), _F32(i - 1))
                d2[pl.ds(c * 16, 16)] = jnp.where(b1, _F32(j), _F32(j - 1))
                d3[pl.ds(c * 16, 16)] = 1.0 + jnp.where(b1, d3u, jnp.where(b2, d3l, d3g))

        ii = _splat_i32(7)
        jj = _splat_i32(7)
        for tt in range(16):
            pp[pl.ds(tt * 16, 16)] = ii
            pp[pl.ds(256 + tt * 16, 16)] = jj
            valid = ii >= 0
            ci = jnp.clip(ii, 0, 7)
            cj = jnp.clip(jj, 0, 7)
            idx = (ci * 8 + cj) * 16 + lanes
            n1 = plsc.load_gather(d1, [idx]).astype(_I32)
            n2 = plsc.load_gather(d2, [idx]).astype(_I32)
            ii = jnp.where(valid, n1, ii)
            jj = jnp.where(valid, n2, jj)

        pltpu.sync_copy(pp, shp)

    plsc.subcore_barrier()

    @pl.when((cid == 0) & (sid < 4))
    def _():
        pltpu.sync_copy(shp, pp)
        b = sid & 1
        t = 1 + (sid >> 1)
        xlane = (t - 1) * 2 + b
        ylane = 4 + xlane
        yj_raw = plsc.load_gather(pp, [lanes * 16 + ylane])
        yb_raw = plsc.load_gather(pp, [256 + lanes * 16 + ylane])
        ym = yj_raw >= 0
        jdx1 = b * 1024 + jnp.clip(yj_raw, 0, 7) * 16
        jdx2 = (t * 2 + b) * 1024 + jnp.clip(yb_raw, 0, 7) * 16

        tvec = zeros
        for p in range(15):
            xr = plsc.load_gather(pp, [_splat_i32(p * 16 + xlane)])
            ar = plsc.load_gather(pp, [_splat_i32(256 + p * 16 + xlane)])
            xm = xr >= 0
            idx1 = jdx1 + jnp.clip(xr, 0, 7) * 128
            idx2 = jdx2 + jnp.clip(ar, 0, 7) * 128
            accq = zeros
            for ch in range(16):
                g1 = plsc.load_gather(fdv, [idx1 + ch])
                g2 = plsc.load_gather(fdv, [idx2 + ch])
                dv = g1 - g2
                accq = accq + dv * dv
            ddq = _vsqrt(accq)
            tvec = tvec + jnp.where(xm & ym, ddq, zeros)

        total = hsum_full(tvec)
        xr_all = plsc.load_gather(pp, [lanes * 16 + xlane])
        cx = hsum_full(jnp.where(xr_all >= 0, _F32(1.0), _F32(0.0)))
        cy = hsum_full(jnp.where(ym, _F32(1.0), _F32(0.0)))
        outv[...] = total / (cx * cy)
        pltpu.sync_copy(outv, shd.at[pl.ds(sid * 16, 16)])

    plsc.subcore_barrier()

    @pl.when((cid == 0) & (sid == 0))
    def _():
        pltpu.sync_copy(shd, hs.at[pl.ds(0, 64)])
        dp0 = hs[pl.ds(0, 16)]
        dp1 = hs[pl.ds(16, 16)]
        dn0 = hs[pl.ds(32, 16)]
        dn1 = hs[pl.ds(48, 16)]
        loss = (jnp.maximum(dp0 - dn0 + _F32(0.1), zeros)
                + jnp.maximum(dp1 - dn1 + _F32(0.1), zeros))
        outv[...] = loss
        pltpu.sync_copy(outv.at[pl.ds(0, 1)], out_hbm)


@jax.jit
def _run(fd_flat):
    mesh = plsc.VectorSubcoreMesh(core_axis_name="c", subcore_axis_name="s")
    k = functools.partial(
        pl.kernel,
        mesh=mesh,
        out_type=jax.ShapeDtypeStruct((1,), _F32),
        compiler_params=pltpu.CompilerParams(needs_layout_passes=False),
        scratch_types=[
            pltpu.VMEM((6144,), _F32),
            pltpu.VMEM((1024,), _F32),
            pltpu.VMEM((1024,), _F32),
            pltpu.VMEM((1024,), _F32),
            pltpu.VMEM((1024,), _F32),
            pltpu.VMEM((1024,), _F32),
            pltpu.VMEM((512,), _I32),
            pltpu.VMEM((256,), _F32),
            pltpu.VMEM((1024,), _F32),
            pltpu.VMEM((16,), _F32),
            pltpu.VMEM_SHARED((512,), _F32),
            pltpu.VMEM_SHARED((512,), _I32),
            pltpu.VMEM_SHARED((64,), _F32),
            pltpu.SemaphoreType.DMA,
        ],
    )(_sc_body)
    return k(fd_flat)


def kernel(feature_data):
    fd_flat = jnp.asarray(feature_data, dtype=_F32).reshape(6144)
    return _run(fd_flat)


# speedup vs baseline: 1.2382x; 1.2382x over previous
import functools

import jax
import jax.numpy as jnp
from jax import lax
from jax.experimental import pallas as pl
from jax.experimental.pallas import tpu as pltpu
from jax.experimental.pallas import tpu_sc as plsc

_F32 = jnp.float32
_I32 = jnp.int32


def _vsqrt(x):
    xi = lax.bitcast_convert_type(x, _I32)
    yi = (xi >> 1) + jnp.int32(0x1FBD1DF5)
    y = lax.bitcast_convert_type(yi, _F32)
    for _ in range(4):
        y = 0.5 * (y + x / y)
    return y


def _splat_i32(v):
    return jnp.zeros((16,), _I32) + v


def _sc_body(fd_hbm, out_hbm, fdv, d0, d1, d2, d3, rr, pp, hs, mcol,
             outv, shm, shp, shd, sem):
    cid = lax.axis_index("c")
    sid = lax.axis_index("s")
    lanes = lax.iota(_I32, 16)
    zeros = jnp.zeros((16,), _F32)

    def hsum_full(x):
        for s in (8, 4, 2, 1):
            hs[pl.ds(0, 16)] = x
            x = x + plsc.load_gather(hs, [lanes ^ s])
        return x

    @pl.when((cid == 0) & (sid < 8))
    def _():
        pltpu.sync_copy(fd_hbm, fdv)
        p = sid
        dirx = p < 4
        tb = p % 4
        t = 1 + tb // 2
        b = tb % 2
        rowmul = jnp.where(dirx, 128, 16)
        wmul = jnp.where(dirx, 16, 128)
        scale = jnp.where(dirx, _F32(0.125), _F32(1.0))
        base1 = b * 1024
        base2 = (t * 2 + b) * 1024

        def m_group(g, _):
            for k in range(16):
                cell = g * 16 + k
                i = cell // 8
                j = cell % 8
                offa = base1 + i * rowmul
                offb = base2 + j * rowmul
                acc = zeros
                for w in range(8):
                    av = fdv[pl.ds(offa + w * wmul, 16)]
                    bv = fdv[pl.ds(offb + w * wmul, 16)]
                    dv = av - bv
                    acc = acc + dv * dv
                hs[pl.ds(k * 16, 16)] = acc
            totals = zeros
            for ch in range(16):
                totals = totals + plsc.load_gather(hs, [lanes * 16 + ch])
            mvec = _vsqrt(totals * scale) + _F32(1e-08)
            mcol[pl.ds(g * 16, 16)] = mvec
            return 0

        lax.fori_loop(0, 4, m_group, 0)
        pltpu.sync_copy(mcol.at[pl.ds(0, 64)], shm.at[pl.ds(p * 64, 64)])

    plsc.subcore_barrier()

    @pl.when((cid == 0) & (sid == 0))
    def _():
        pltpu.sync_copy(shm, mcol.at[pl.ds(0, 512)])
        l64 = lanes * 64

        def mrow(c):
            return plsc.load_gather(mcol, [l64 + c])

        m0 = mrow(0)
        d0[pl.ds(0, 16)] = m0
        d1[pl.ds(0, 16)] = zeros - 1.0
        d2[pl.ds(0, 16)] = zeros - 1.0
        d3[pl.ds(0, 16)] = zeros + 1.0
        rr[pl.ds(0, 16)] = m0

        def i_edge(i, _):
            c = i * 8
            v0 = mrow(c) + d0[pl.ds((c - 8) * 16, 16)]
            v3 = d3[pl.ds((c - 8) * 16, 16)] + 1.0
            d0[pl.ds(c * 16, 16)] = v0
            d1[pl.ds(c * 16, 16)] = zeros + (i - 1).astype(_F32)
            d2[pl.ds(c * 16, 16)] = zeros
            d3[pl.ds(c * 16, 16)] = v3
            rr[pl.ds(c * 16, 16)] = v0 / v3
            return 0

        lax.fori_loop(1, 8, i_edge, 0)

        def j_edge(j, _):
            v0 = mrow(j) + d0[pl.ds((j - 1) * 16, 16)]
            v3 = d3[pl.ds((j - 1) * 16, 16)] + 1.0
            d0[pl.ds(j * 16, 16)] = v0
            d1[pl.ds(j * 16, 16)] = zeros
            d2[pl.ds(j * 16, 16)] = zeros + (j - 1).astype(_F32)
            d3[pl.ds(j * 16, 16)] = v3
            rr[pl.ds(j * 16, 16)] = v0 / v3
            return 0

        lax.fori_loop(1, 8, j_edge, 0)

        def dp_cell(q, _):
            i = 1 + q // 7
            j = 1 + q % 7
            c = i * 8 + j
            c1 = rr[pl.ds((c - 8) * 16, 16)]
            c2 = rr[pl.ds((c - 1) * 16, 16)]
            c3 = rr[pl.ds((c - 9) * 16, 16)]
            b1 = (c1 <= c2) & (c1 <= c3)
            b2 = c2 <= c3
            d0u = d0[pl.ds((c - 8) * 16, 16)]
            d0l = d0[pl.ds((c - 1) * 16, 16)]
            d0g = d0[pl.ds((c - 9) * 16, 16)]
            d3u = d3[pl.ds((c - 8) * 16, 16)]
            d3l = d3[pl.ds((c - 1) * 16, 16)]
            d3g = d3[pl.ds((c - 9) * 16, 16)]
            v0 = mrow(c) + jnp.where(b1, d0u, jnp.where(b2, d0l, d0g))
            v3 = 1.0 + jnp.where(b1, d3u, jnp.where(b2, d3l, d3g))
            fi = i.astype(_F32)
            fj = j.astype(_F32)
            d0[pl.ds(c * 16, 16)] = v0
            d1[pl.ds(c * 16, 16)] = jnp.where((~b1) & b2, fi, fi - 1.0)
            d2[pl.ds(c * 16, 16)] = jnp.where(b1, fj, fj - 1.0)
            d3[pl.ds(c * 16, 16)] = v3
            rr[pl.ds(c * 16, 16)] = v0 / v3
            return 0

        lax.fori_loop(0, 49, dp_cell, 0)

        def bt_step(tt, carry):
            ii, jj = carry
            pp[pl.ds(tt * 16, 16)] = ii
            pp[pl.ds(256 + tt * 16, 16)] = jj
            valid = ii >= 0
            ci = jnp.clip(ii, 0, 7)
            cj = jnp.clip(jj, 0, 7)
            idx = (ci * 8 + cj) * 16 + lanes
            n1 = plsc.load_gather(d1, [idx]).astype(_I32)
            n2 = plsc.load_gather(d2, [idx]).astype(_I32)
            return (jnp.where(valid, n1, ii), jnp.where(valid, n2, jj))

        seven = _splat_i32(7)
        lax.fori_loop(0, 16, bt_step, (seven, seven))
        pltpu.sync_copy(pp, shp)

    plsc.subcore_barrier()

    @pl.when((cid == 0) & (sid < 4))
    def _():
        pltpu.sync_copy(shp, pp)
        b = sid & 1
        t = 1 + (sid >> 1)
        xlane = (t - 1) * 2 + b
        ylane = 4 + xlane
        yj_raw = plsc.load_gather(pp, [lanes * 16 + ylane])
        yb_raw = plsc.load_gather(pp, [256 + lanes * 16 + ylane])
        ym = yj_raw >= 0
        jdx1 = b * 1024 + jnp.clip(yj_raw, 0, 7) * 16
        jdx2 = (t * 2 + b) * 1024 + jnp.clip(yb_raw, 0, 7) * 16

        def p_body(p, tv):
            xr = plsc.load_gather(pp, [_splat_i32(p * 16 + xlane)])
            ar = plsc.load_gather(pp, [_splat_i32(256 + p * 16 + xlane)])
            xm = xr >= 0
            idx1 = jdx1 + jnp.clip(xr, 0, 7) * 128
            idx2 = jdx2 + jnp.clip(ar, 0, 7) * 128
            accq = zeros
            for ch in range(16):
                g1 = plsc.load_gather(fdv, [idx1 + ch])
                g2 = plsc.load_gather(fdv, [idx2 + ch])
                dv = g1 - g2
                accq = accq + dv * dv
            ddq = _vsqrt(accq)
            return tv + jnp.where(xm & ym, ddq, zeros)

        tvec = lax.fori_loop(0, 15, p_body, zeros)
        total = hsum_full(tvec)
        xr_all = plsc.load_gather(pp, [lanes * 16 + xlane])
        cx = hsum_full(jnp.where(xr_all >= 0, _F32(1.0), _F32(0.0)))
        cy = hsum_full(jnp.where(ym, _F32(1.0), _F32(0.0)))
        outv[...] = total / (cx * cy)
        pltpu.sync_copy(outv, shd.at[pl.ds(sid * 16, 16)])

    plsc.subcore_barrier()

    @pl.when((cid == 0) & (sid == 0))
    def _():
        pltpu.sync_copy(shd, hs.at[pl.ds(0, 64)])
        dp0 = hs[pl.ds(0, 16)]
        dp1 = hs[pl.ds(16, 16)]
        dn0 = hs[pl.ds(32, 16)]
        dn1 = hs[pl.ds(48, 16)]
        loss = (jnp.maximum(dp0 - dn0 + _F32(0.1), zeros)
                + jnp.maximum(dp1 - dn1 + _F32(0.1), zeros))
        outv[...] = loss
        pltpu.sync_copy(outv.at[pl.ds(0, 1)], out_hbm)


@jax.jit
def _run(fd_flat):
    mesh = plsc.VectorSubcoreMesh(core_axis_name="c", subcore_axis_name="s")
    k = functools.partial(
        pl.kernel,
        mesh=mesh,
        out_type=jax.ShapeDtypeStruct((1,), _F32),
        compiler_params=pltpu.CompilerParams(needs_layout_passes=False),
        scratch_types=[
            pltpu.VMEM((6144,), _F32),
            pltpu.VMEM((1024,), _F32),
            pltpu.VMEM((1024,), _F32),
            pltpu.VMEM((1024,), _F32),
            pltpu.VMEM((1024,), _F32),
            pltpu.VMEM((1024,), _F32),
            pltpu.VMEM((512,), _I32),
            pltpu.VMEM((256,), _F32),
            pltpu.VMEM((1024,), _F32),
            pltpu.VMEM((16,), _F32),
            pltpu.VMEM_SHARED((512,), _F32),
            pltpu.VMEM_SHARED((512,), _I32),
            pltpu.VMEM_SHARED((64,), _F32),
            pltpu.SemaphoreType.DMA,
        ],
    )(_sc_body)
    return k(fd_flat)


def kernel(feature_data):
    fd_flat = jnp.asarray(feature_data, dtype=_F32).reshape(6144)
    return _run(fd_flat)


# speedup vs baseline: 1.2391x; 1.0007x over previous
import functools

import jax
import jax.numpy as jnp
from jax import lax
from jax.experimental import pallas as pl
from jax.experimental.pallas import tpu as pltpu
from jax.experimental.pallas import tpu_sc as plsc

_F32 = jnp.float32
_I32 = jnp.int32


def _vsqrt(x):
    xi = lax.bitcast_convert_type(x, _I32)
    yi = (xi >> 1) + jnp.int32(0x1FBD1DF5)
    y = lax.bitcast_convert_type(yi, _F32)
    for _ in range(4):
        y = 0.5 * (y + x / y)
    return y


def _splat_i32(v):
    return jnp.zeros((16,), _I32) + v


def _sc_body(fd_hbm, out_hbm, fdv, d0, d1, d2, d3, rr, pp, hs, mcol,
             outv, shm, shp, shd, sem):
    cid = lax.axis_index("c")
    sid = lax.axis_index("s")
    lanes = lax.iota(_I32, 16)
    zeros = jnp.zeros((16,), _F32)

    def hsum_full(x):
        for s in (8, 4, 2, 1):
            hs[pl.ds(0, 16)] = x
            x = x + plsc.load_gather(hs, [lanes ^ s])
        return x

    @pl.when((cid == 0) & (sid < 8))
    def _():
        pltpu.sync_copy(fd_hbm, fdv)
        p = sid
        dirx = p < 4
        tb = p % 4
        t = 1 + tb // 2
        b = tb % 2
        rowmul = jnp.where(dirx, 128, 16)
        wmul = jnp.where(dirx, 16, 128)
        scale = jnp.where(dirx, _F32(0.125), _F32(1.0))
        base1 = b * 1024
        base2 = (t * 2 + b) * 1024

        colb = [base2 + j * rowmul for j in range(8)]
        woff = [w * wmul for w in range(8)]

        def m_group(g, _):
            ga = base1 + (g * 2) * rowmul
            oas = (ga, ga + rowmul)
            for k in range(16):
                oa = oas[k // 8]
                ob = colb[k % 8]
                acc = zeros
                for w in range(8):
                    av = fdv[pl.ds(oa + woff[w], 16)]
                    bv = fdv[pl.ds(ob + woff[w], 16)]
                    dv = av - bv
                    acc = acc + dv * dv
                hs[pl.ds(k * 16, 16)] = acc
            totals = zeros
            for ch in range(16):
                totals = totals + plsc.load_gather(hs, [lanes * 16 + ch])
            mvec = _vsqrt(totals * scale) + _F32(1e-08)
            mcol[pl.ds(g * 16, 16)] = mvec
            return 0

        lax.fori_loop(0, 4, m_group, 0)
        pltpu.sync_copy(mcol.at[pl.ds(0, 64)], shm.at[pl.ds(p * 64, 64)])

    plsc.subcore_barrier()

    @pl.when((cid == 0) & (sid == 0))
    def _():
        pltpu.sync_copy(shm, mcol.at[pl.ds(0, 512)])
        l64 = lanes * 64

        def mrow(c):
            return plsc.load_gather(mcol, [l64 + c])

        m0 = mrow(0)
        d0[pl.ds(0, 16)] = m0
        d1[pl.ds(0, 16)] = zeros - 1.0
        d2[pl.ds(0, 16)] = zeros - 1.0
        d3[pl.ds(0, 16)] = zeros + 1.0
        rr[pl.ds(0, 16)] = m0

        def i_edge(i, _):
            c = i * 8
            v0 = mrow(c) + d0[pl.ds((c - 8) * 16, 16)]
            v3 = d3[pl.ds((c - 8) * 16, 16)] + 1.0
            d0[pl.ds(c * 16, 16)] = v0
            d1[pl.ds(c * 16, 16)] = zeros + (i - 1).astype(_F32)
            d2[pl.ds(c * 16, 16)] = zeros
            d3[pl.ds(c * 16, 16)] = v3
            rr[pl.ds(c * 16, 16)] = v0 / v3
            return 0

        lax.fori_loop(1, 8, i_edge, 0)

        def j_edge(j, _):
            v0 = mrow(j) + d0[pl.ds((j - 1) * 16, 16)]
            v3 = d3[pl.ds((j - 1) * 16, 16)] + 1.0
            d0[pl.ds(j * 16, 16)] = v0
            d1[pl.ds(j * 16, 16)] = zeros
            d2[pl.ds(j * 16, 16)] = zeros + (j - 1).astype(_F32)
            d3[pl.ds(j * 16, 16)] = v3
            rr[pl.ds(j * 16, 16)] = v0 / v3
            return 0

        lax.fori_loop(1, 8, j_edge, 0)

        def dp_row(i, _):
            base = i * 128
            fi = zeros + i.astype(_F32)
            lcar = (d0[pl.ds(base, 16)], d3[pl.ds(base, 16)],
                    rr[pl.ds(base, 16)])

            def dp_cell(j, lcar):
                d0l, d3l, c2 = lcar
                c16 = base + j * 16
                c1 = rr[pl.ds(c16 - 128, 16)]
                c3 = rr[pl.ds(c16 - 144, 16)]
                b1 = (c1 <= c2) & (c1 <= c3)
                b2 = c2 <= c3
                d0u = d0[pl.ds(c16 - 128, 16)]
                d0g = d0[pl.ds(c16 - 144, 16)]
                d3u = d3[pl.ds(c16 - 128, 16)]
                d3g = d3[pl.ds(c16 - 144, 16)]
                v0 = plsc.load_gather(mcol, [l64 + (c16 >> 4)]) + jnp.where(
                    b1, d0u, jnp.where(b2, d0l, d0g))
                v3 = 1.0 + jnp.where(b1, d3u, jnp.where(b2, d3l, d3g))
                fj = zeros + j.astype(_F32)
                vr = v0 / v3
                d0[pl.ds(c16, 16)] = v0
                d1[pl.ds(c16, 16)] = jnp.where((~b1) & b2, fi, fi - 1.0)
                d2[pl.ds(c16, 16)] = jnp.where(b1, fj, fj - 1.0)
                d3[pl.ds(c16, 16)] = v3
                rr[pl.ds(c16, 16)] = vr
                return (v0, v3, vr)

            lax.fori_loop(1, 8, dp_cell, lcar)
            return 0

        lax.fori_loop(1, 8, dp_row, 0)

        def bt_step(tt, carry):
            ii, jj = carry
            pp[pl.ds(tt * 16, 16)] = ii
            pp[pl.ds(256 + tt * 16, 16)] = jj
            valid = ii >= 0
            ci = jnp.clip(ii, 0, 7)
            cj = jnp.clip(jj, 0, 7)
            idx = (ci * 8 + cj) * 16 + lanes
            n1 = plsc.load_gather(d1, [idx]).astype(_I32)
            n2 = plsc.load_gather(d2, [idx]).astype(_I32)
            return (jnp.where(valid, n1, ii), jnp.where(valid, n2, jj))

        seven = _splat_i32(7)
        lax.fori_loop(0, 16, bt_step, (seven, seven))
        pltpu.sync_copy(pp, shp)

    plsc.subcore_barrier()

    @pl.when((cid == 0) & (sid < 4))
    def _():
        pltpu.sync_copy(shp, pp)
        b = sid & 1
        t = 1 + (sid >> 1)
        xlane = (t - 1) * 2 + b
        ylane = 4 + xlane
        yj_raw = plsc.load_gather(pp, [lanes * 16 + ylane])
        yb_raw = plsc.load_gather(pp, [256 + lanes * 16 + ylane])
        ym = yj_raw >= 0
        jdx1 = b * 1024 + jnp.clip(yj_raw, 0, 7) * 16
        jdx2 = (t * 2 + b) * 1024 + jnp.clip(yb_raw, 0, 7) * 16

        def p_body(p, tv):
            xr = plsc.load_gather(pp, [_splat_i32(p * 16 + xlane)])
            ar = plsc.load_gather(pp, [_splat_i32(256 + p * 16 + xlane)])
            xm = xr >= 0
            idx1 = jdx1 + jnp.clip(xr, 0, 7) * 128
            idx2 = jdx2 + jnp.clip(ar, 0, 7) * 128
            accq = zeros
            for ch in range(16):
                g1 = plsc.load_gather(fdv, [idx1 + ch])
                g2 = plsc.load_gather(fdv, [idx2 + ch])
                dv = g1 - g2
                accq = accq + dv * dv
            ddq = _vsqrt(accq)
            return tv + jnp.where(xm & ym, ddq, zeros)

        tvec = lax.fori_loop(0, 15, p_body, zeros)
        total = hsum_full(tvec)
        xr_all = plsc.load_gather(pp, [lanes * 16 + xlane])
        cx = hsum_full(jnp.where(xr_all >= 0, _F32(1.0), _F32(0.0)))
        cy = hsum_full(jnp.where(ym, _F32(1.0), _F32(0.0)))
        outv[...] = total / (cx * cy)
        pltpu.sync_copy(outv, shd.at[pl.ds(sid * 16, 16)])

    plsc.subcore_barrier()

    @pl.when((cid == 0) & (sid == 0))
    def _():
        pltpu.sync_copy(shd, hs.at[pl.ds(0, 64)])
        dp0 = hs[pl.ds(0, 16)]
        dp1 = hs[pl.ds(16, 16)]
        dn0 = hs[pl.ds(32, 16)]
        dn1 = hs[pl.ds(48, 16)]
        loss = (jnp.maximum(dp0 - dn0 + _F32(0.1), zeros)
                + jnp.maximum(dp1 - dn1 + _F32(0.1), zeros))
        outv[...] = loss
        pltpu.sync_copy(outv.at[pl.ds(0, 1)], out_hbm)


@jax.jit
def _run(fd_flat):
    mesh = plsc.VectorSubcoreMesh(core_axis_name="c", subcore_axis_name="s")
    k = functools.partial(
        pl.kernel,
        mesh=mesh,
        out_type=jax.ShapeDtypeStruct((1,), _F32),
        compiler_params=pltpu.CompilerParams(needs_layout_passes=False),
        scratch_types=[
            pltpu.VMEM((6144,), _F32),
            pltpu.VMEM((1024,), _F32),
            pltpu.VMEM((1024,), _F32),
            pltpu.VMEM((1024,), _F32),
            pltpu.VMEM((1024,), _F32),
            pltpu.VMEM((1024,), _F32),
            pltpu.VMEM((512,), _I32),
            pltpu.VMEM((256,), _F32),
            pltpu.VMEM((1024,), _F32),
            pltpu.VMEM((16,), _F32),
            pltpu.VMEM_SHARED((512,), _F32),
            pltpu.VMEM_SHARED((512,), _I32),
            pltpu.VMEM_SHARED((64,), _F32),
            pltpu.SemaphoreType.DMA,
        ],
    )(_sc_body)
    return k(fd_flat)


def kernel(feature_data):
    fd_flat = jnp.asarray(feature_data, dtype=_F32).reshape(6144)
    return _run(fd_flat)


# speedup vs baseline: 1.2441x; 1.0040x over previous
import functools

import jax
import jax.numpy as jnp
from jax import lax
from jax.experimental import pallas as pl
from jax.experimental.pallas import tpu as pltpu
from jax.experimental.pallas import tpu_sc as plsc

_F32 = jnp.float32
_I32 = jnp.int32


def _vsqrt(x):
    xi = lax.bitcast_convert_type(x, _I32)
    yi = jnp.int32(0x5F3759DF) - (xi >> 1)
    y = lax.bitcast_convert_type(yi, _F32)
    for _ in range(4):
        y = y * (1.5 - 0.5 * x * y * y)
    return x * y


def _splat_i32(v):
    return jnp.zeros((16,), _I32) + v


def _sc_body(fd_hbm, out_hbm, fdv, d0, d1, d2, d3, rr, pp, hs, mcol,
             outv, shm, shp, shd, sem):
    cid = lax.axis_index("c")
    sid = lax.axis_index("s")
    lanes = lax.iota(_I32, 16)
    zeros = jnp.zeros((16,), _F32)

    def hsum_full(x):
        for s in (8, 4, 2, 1):
            hs[pl.ds(240, 16)] = x
            x = x + plsc.load_gather(hs, [240 + (lanes ^ s)])
        return x

    @pl.when((cid == 0) & (sid < 8))
    def _():
        p = sid
        dirx = p < 4
        tb = p % 4
        t = 1 + tb // 2
        b = tb % 2
        rowmul = jnp.where(dirx, 128, 16)
        wmul = jnp.where(dirx, 16, 128)
        scale = jnp.where(dirx, _F32(0.125), _F32(1.0))
        base1 = b * 1024
        base2 = (t * 2 + b) * 1024
        h1 = pltpu.async_copy(fd_hbm.at[pl.ds(base1, 1024)],
                              fdv.at[pl.ds(base1, 1024)], sem)
        h2 = pltpu.async_copy(fd_hbm.at[pl.ds(base2, 1024)],
                              fdv.at[pl.ds(base2, 1024)], sem)
        h1.wait()
        h2.wait()

        colb = [base2 + j * rowmul for j in range(8)]
        woff = [w * wmul for w in range(8)]

        def m_group(g, _):
            ga = base1 + (g * 2) * rowmul
            oas = (ga, ga + rowmul)
            for k in range(16):
                oa = oas[k // 8]
                ob = colb[k % 8]
                acc = zeros
                for w in range(8):
                    av = fdv[pl.ds(oa + woff[w], 16)]
                    bv = fdv[pl.ds(ob + woff[w], 16)]
                    dv = av - bv
                    acc = acc + dv * dv
                hs[pl.ds(k * 16, 16)] = acc
            totals = zeros
            for ch in range(16):
                totals = totals + plsc.load_gather(hs, [lanes * 16 + ch])
            mvec = _vsqrt(totals * scale) + _F32(1e-08)
            mcol[pl.ds(g * 16, 16)] = mvec
            return 0

        lax.fori_loop(0, 4, m_group, 0)
        pltpu.sync_copy(mcol.at[pl.ds(0, 64)], shm.at[pl.ds(p * 64, 64)])

    plsc.subcore_barrier()

    @pl.when((cid == 0) & (sid == 0))
    def _():
        pltpu.sync_copy(shm, mcol.at[pl.ds(0, 512)])
        l64 = lanes * 64

        def mrow(c):
            return plsc.load_gather(mcol, [l64 + c])

        m0 = mrow(0)
        d0[pl.ds(0, 16)] = m0
        d1[pl.ds(0, 16)] = zeros - 1.0
        d2[pl.ds(0, 16)] = zeros - 1.0
        d3[pl.ds(0, 16)] = zeros + 1.0
        rr[pl.ds(0, 16)] = m0

        def i_edge(i, _):
            c = i * 8
            v0 = mrow(c) + d0[pl.ds((c - 8) * 16, 16)]
            v3 = d3[pl.ds((c - 8) * 16, 16)] + 1.0
            d0[pl.ds(c * 16, 16)] = v0
            d1[pl.ds(c * 16, 16)] = zeros + (i - 1).astype(_F32)
            d2[pl.ds(c * 16, 16)] = zeros
            d3[pl.ds(c * 16, 16)] = v3
            rr[pl.ds(c * 16, 16)] = v0 / v3
            return 0

        lax.fori_loop(1, 8, i_edge, 0)

        def j_edge(j, _):
            v0 = mrow(j) + d0[pl.ds((j - 1) * 16, 16)]
            v3 = d3[pl.ds((j - 1) * 16, 16)] + 1.0
            d0[pl.ds(j * 16, 16)] = v0
            d1[pl.ds(j * 16, 16)] = zeros
            d2[pl.ds(j * 16, 16)] = zeros + (j - 1).astype(_F32)
            d3[pl.ds(j * 16, 16)] = v3
            rr[pl.ds(j * 16, 16)] = v0 / v3
            return 0

        lax.fori_loop(1, 8, j_edge, 0)

        def dp_row(i, _):
            base = i * 128
            fi = zeros + i.astype(_F32)
            lcar = (d0[pl.ds(base, 16)], d3[pl.ds(base, 16)],
                    rr[pl.ds(base, 16)])

            def dp_cell(j, lcar):
                d0l, d3l, c2 = lcar
                c16 = base + j * 16
                c1 = rr[pl.ds(c16 - 128, 16)]
                c3 = rr[pl.ds(c16 - 144, 16)]
                b1 = (c1 <= c2) & (c1 <= c3)
                b2 = c2 <= c3
                d0u = d0[pl.ds(c16 - 128, 16)]
                d0g = d0[pl.ds(c16 - 144, 16)]
                d3u = d3[pl.ds(c16 - 128, 16)]
                d3g = d3[pl.ds(c16 - 144, 16)]
                v0 = plsc.load_gather(mcol, [l64 + (c16 >> 4)]) + jnp.where(
                    b1, d0u, jnp.where(b2, d0l, d0g))
                v3 = 1.0 + jnp.where(b1, d3u, jnp.where(b2, d3l, d3g))
                fj = zeros + j.astype(_F32)
                vr = v0 / v3
                d0[pl.ds(c16, 16)] = v0
                d1[pl.ds(c16, 16)] = jnp.where((~b1) & b2, fi, fi - 1.0)
                d2[pl.ds(c16, 16)] = jnp.where(b1, fj, fj - 1.0)
                d3[pl.ds(c16, 16)] = v3
                rr[pl.ds(c16, 16)] = vr
                return (v0, v3, vr)

            lax.fori_loop(1, 8, dp_cell, lcar)
            return 0

        lax.fori_loop(1, 8, dp_row, 0)

        def bt_step(tt, carry):
            ii, jj = carry
            pp[pl.ds(tt * 16, 16)] = ii
            pp[pl.ds(256 + tt * 16, 16)] = jj
            valid = ii >= 0
            ci = jnp.clip(ii, 0, 7)
            cj = jnp.clip(jj, 0, 7)
            idx = (ci * 8 + cj) * 16 + lanes
            n1 = plsc.load_gather(d1, [idx]).astype(_I32)
            n2 = plsc.load_gather(d2, [idx]).astype(_I32)
            return (jnp.where(valid, n1, ii), jnp.where(valid, n2, jj))

        seven = _splat_i32(7)
        lax.fori_loop(0, 16, bt_step, (seven, seven))
        pltpu.sync_copy(pp, shp)

    plsc.subcore_barrier()

    @pl.when((cid == 0) & (sid < 4))
    def _():
        pltpu.sync_copy(shp, pp)
        b = sid & 1
        t = 1 + (sid >> 1)
        xlane = (t - 1) * 2 + b
        ylane = 4 + xlane
        yj_raw = plsc.load_gather(pp, [lanes * 16 + ylane])
        yb_raw = plsc.load_gather(pp, [256 + lanes * 16 + ylane])
        ym = yj_raw >= 0
        jdx1 = b * 1024 + jnp.clip(yj_raw, 0, 7) * 16
        jdx2 = (t * 2 + b) * 1024 + jnp.clip(yb_raw, 0, 7) * 16

        def p_store(p):
            xr = plsc.load_gather(pp, [_splat_i32(p * 16 + xlane)])
            ar = plsc.load_gather(pp, [_splat_i32(256 + p * 16 + xlane)])
            xm = xr >= 0
            idx1 = jdx1 + jnp.clip(xr, 0, 7) * 128
            idx2 = jdx2 + jnp.clip(ar, 0, 7) * 128
            accq = zeros
            for ch in range(16):
                g1 = plsc.load_gather(fdv, [idx1 + ch])
                g2 = plsc.load_gather(fdv, [idx2 + ch])
                dv = g1 - g2
                accq = accq + dv * dv
            hs[pl.ds(p * 16, 16)] = jnp.where(xm & ym, accq, zeros)

        def p_pair(q, _):
            p_store(q * 2)
            p_store(q * 2 + 1)
            return 0

        lax.fori_loop(0, 7, p_pair, 0)
        p_store(14)
        tvec = zeros
        for p in range(15):
            tvec = tvec + _vsqrt(hs[pl.ds(p * 16, 16)])
        total = hsum_full(tvec)
        xr_all = plsc.load_gather(pp, [lanes * 16 + xlane])
        cx = hsum_full(jnp.where(xr_all >= 0, _F32(1.0), _F32(0.0)))
        cy = hsum_full(jnp.where(ym, _F32(1.0), _F32(0.0)))
        outv[...] = total / (cx * cy)
        pltpu.sync_copy(outv, shd.at[pl.ds(sid * 16, 16)])

    plsc.subcore_barrier()

    @pl.when((cid == 0) & (sid == 0))
    def _():
        pltpu.sync_copy(shd, hs.at[pl.ds(0, 64)])
        dp0 = hs[pl.ds(0, 16)]
        dp1 = hs[pl.ds(16, 16)]
        dn0 = hs[pl.ds(32, 16)]
        dn1 = hs[pl.ds(48, 16)]
        loss = (jnp.maximum(dp0 - dn0 + _F32(0.1), zeros)
                + jnp.maximum(dp1 - dn1 + _F32(0.1), zeros))
        outv[...] = loss
        pltpu.sync_copy(outv.at[pl.ds(0, 1)], out_hbm)


@jax.jit
def _run(fd_flat):
    mesh = plsc.VectorSubcoreMesh(core_axis_name="c", subcore_axis_name="s")
    k = functools.partial(
        pl.kernel,
        mesh=mesh,
        out_type=jax.ShapeDtypeStruct((1,), _F32),
        compiler_params=pltpu.CompilerParams(needs_layout_passes=False),
        scratch_types=[
            pltpu.VMEM((6144,), _F32),
            pltpu.VMEM((1024,), _F32),
            pltpu.VMEM((1024,), _F32),
            pltpu.VMEM((1024,), _F32),
            pltpu.VMEM((1024,), _F32),
            pltpu.VMEM((1024,), _F32),
            pltpu.VMEM((512,), _I32),
            pltpu.VMEM((256,), _F32),
            pltpu.VMEM((1024,), _F32),
            pltpu.VMEM((16,), _F32),
            pltpu.VMEM_SHARED((512,), _F32),
            pltpu.VMEM_SHARED((512,), _I32),
            pltpu.VMEM_SHARED((64,), _F32),
            pltpu.SemaphoreType.DMA,
        ],
    )(_sc_body)
    return k(fd_flat)


def kernel(feature_data):
    fd_flat = jnp.asarray(feature_data, dtype=_F32).reshape(6144)
    return _run(fd_flat)


# speedup vs baseline: 1.3199x; 1.0609x over previous
import functools

import jax
import jax.numpy as jnp
from jax import lax
from jax.experimental import pallas as pl
from jax.experimental.pallas import tpu as pltpu
from jax.experimental.pallas import tpu_sc as plsc

_F32 = jnp.float32
_I32 = jnp.int32


def _vsqrt(x):
    xi = lax.bitcast_convert_type(x, _I32)
    yi = jnp.int32(0x5F3759DF) - (xi >> 1)
    y = lax.bitcast_convert_type(yi, _F32)
    for _ in range(4):
        y = y * (1.5 - 0.5 * x * y * y)
    return x * y


def _splat_i32(v):
    return jnp.zeros((16,), _I32) + v


def _sc_body(fd_hbm, out_hbm, fdv, d0, d1, d2, d3, rr, pp, hs, mcol,
             outv, shm, shp, shd, sem):
    cid = lax.axis_index("c")
    sid = lax.axis_index("s")
    lanes = lax.iota(_I32, 16)
    zeros = jnp.zeros((16,), _F32)

    def hsum_full(x):
        for s in (8, 4, 2, 1):
            hs[pl.ds(240, 16)] = x
            x = x + plsc.load_gather(hs, [240 + (lanes ^ s)])
        return x

    @pl.when((cid == 0) & (sid < 8))
    def _():
        p = sid
        dirx = p < 4
        tb = p % 4
        t = 1 + tb // 2
        b = tb % 2
        rowmul = jnp.where(dirx, 128, 16)
        wmul = jnp.where(dirx, 16, 128)
        scale = jnp.where(dirx, _F32(0.125), _F32(1.0))
        base1 = b * 1024
        base2 = (t * 2 + b) * 1024
        h1 = pltpu.async_copy(fd_hbm.at[pl.ds(base1, 1024)],
                              fdv.at[pl.ds(base1, 1024)], sem)
        h2 = pltpu.async_copy(fd_hbm.at[pl.ds(base2, 1024)],
                              fdv.at[pl.ds(base2, 1024)], sem)
        h1.wait()
        h2.wait()

        colb = [base2 + j * rowmul for j in range(8)]
        woff = [w * wmul for w in range(8)]

        def m_group(g, _):
            ga = base1 + (g * 2) * rowmul
            oas = (ga, ga + rowmul)
            for k in range(16):
                oa = oas[k // 8]
                ob = colb[k % 8]
                acc = zeros
                for w in range(8):
                    av = fdv[pl.ds(oa + woff[w], 16)]
                    bv = fdv[pl.ds(ob + woff[w], 16)]
                    dv = av - bv
                    acc = acc + dv * dv
                hs[pl.ds(k * 16, 16)] = acc
            totals = zeros
            for ch in range(16):
                totals = totals + plsc.load_gather(hs, [lanes * 16 + ch])
            mvec = _vsqrt(totals * scale) + _F32(1e-08)
            mcol[pl.ds(g * 16, 16)] = mvec
            return 0

        lax.fori_loop(0, 4, m_group, 0)
        pltpu.sync_copy(mcol.at[pl.ds(0, 64)], shm.at[pl.ds(p * 64, 64)])

    plsc.subcore_barrier()

    @pl.when((cid == 0) & (sid == 0))
    def _():
        pltpu.sync_copy(shm, mcol.at[pl.ds(0, 512)])
        l64 = lanes * 64

        def mrow(c):
            return plsc.load_gather(mcol, [l64 + c])

        m0 = mrow(0)
        d0[pl.ds(0, 16)] = m0
        d1[pl.ds(0, 16)] = zeros - 1.0
        d2[pl.ds(0, 16)] = zeros - 1.0
        d3[pl.ds(0, 16)] = zeros + 1.0
        rr[pl.ds(0, 16)] = m0

        def i_edge(i, _):
            c = i * 8
            v0 = mrow(c) + d0[pl.ds((c - 8) * 16, 16)]
            v3 = d3[pl.ds((c - 8) * 16, 16)] + 1.0
            d0[pl.ds(c * 16, 16)] = v0
            d1[pl.ds(c * 16, 16)] = zeros + (i - 1).astype(_F32)
            d2[pl.ds(c * 16, 16)] = zeros
            d3[pl.ds(c * 16, 16)] = v3
            rr[pl.ds(c * 16, 16)] = v0 / v3
            return 0

        lax.fori_loop(1, 8, i_edge, 0)

        def j_edge(j, _):
            v0 = mrow(j) + d0[pl.ds((j - 1) * 16, 16)]
            v3 = d3[pl.ds((j - 1) * 16, 16)] + 1.0
            d0[pl.ds(j * 16, 16)] = v0
            d1[pl.ds(j * 16, 16)] = zeros
            d2[pl.ds(j * 16, 16)] = zeros + (j - 1).astype(_F32)
            d3[pl.ds(j * 16, 16)] = v3
            rr[pl.ds(j * 16, 16)] = v0 / v3
            return 0

        lax.fori_loop(1, 8, j_edge, 0)

        def dp_row(i, _):
            base = i * 128
            fi = zeros + i.astype(_F32)
            lcar = (d0[pl.ds(base, 16)], d3[pl.ds(base, 16)],
                    rr[pl.ds(base, 16)])

            def dp_cell(j, lcar):
                d0l, d3l, c2 = lcar
                c16 = base + j * 16
                c1 = rr[pl.ds(c16 - 128, 16)]
                c3 = rr[pl.ds(c16 - 144, 16)]
                b1 = (c1 <= c2) & (c1 <= c3)
                b2 = c2 <= c3
                d0u = d0[pl.ds(c16 - 128, 16)]
                d0g = d0[pl.ds(c16 - 144, 16)]
                d3u = d3[pl.ds(c16 - 128, 16)]
                d3g = d3[pl.ds(c16 - 144, 16)]
                v0 = plsc.load_gather(mcol, [l64 + (c16 >> 4)]) + jnp.where(
                    b1, d0u, jnp.where(b2, d0l, d0g))
                v3 = 1.0 + jnp.where(b1, d3u, jnp.where(b2, d3l, d3g))
                fj = zeros + j.astype(_F32)
                vr = v0 / v3
                d0[pl.ds(c16, 16)] = v0
                d1[pl.ds(c16, 16)] = jnp.where((~b1) & b2, fi, fi - 1.0)
                d2[pl.ds(c16, 16)] = jnp.where(b1, fj, fj - 1.0)
                d3[pl.ds(c16, 16)] = v3
                rr[pl.ds(c16, 16)] = vr
                return (v0, v3, vr)

            lax.fori_loop(1, 8, dp_cell, lcar)
            return 0

        lax.fori_loop(1, 8, dp_row, 0)

        def bt_step(tt, carry):
            ii, jj = carry
            pp[pl.ds(tt * 16, 16)] = ii
            pp[pl.ds(256 + tt * 16, 16)] = jj
            valid = ii >= 0
            ci = jnp.clip(ii, 0, 7)
            cj = jnp.clip(jj, 0, 7)
            idx = (ci * 8 + cj) * 16 + lanes
            n1 = plsc.load_gather(d1, [idx]).astype(_I32)
            n2 = plsc.load_gather(d2, [idx]).astype(_I32)
            return (jnp.where(valid, n1, ii), jnp.where(valid, n2, jj))

        seven = _splat_i32(7)
        lax.fori_loop(0, 16, bt_step, (seven, seven))
        pltpu.sync_copy(pp, shp)

    plsc.subcore_barrier()

    @pl.when((cid == 0) & (sid < 4))
    def _():
        pltpu.sync_copy(shp, pp)
        b = sid & 1
        t = 1 + (sid >> 1)
        xlane = (t - 1) * 2 + b
        ylane = 4 + xlane
        yj_raw = plsc.load_gather(pp, [lanes * 16 + ylane])
        yb_raw = plsc.load_gather(pp, [256 + lanes * 16 + ylane])
        ym = yj_raw >= 0
        jdx1 = b * 1024 + jnp.clip(yj_raw, 0, 7) * 16
        jdx2 = (t * 2 + b) * 1024 + jnp.clip(yb_raw, 0, 7) * 16

        def p_store(p):
            xr = plsc.load_gather(pp, [_splat_i32(p * 16 + xlane)])
            ar = plsc.load_gather(pp, [_splat_i32(256 + p * 16 + xlane)])
            xm = xr >= 0
            idx1 = jdx1 + jnp.clip(xr, 0, 7) * 128
            idx2 = jdx2 + jnp.clip(ar, 0, 7) * 128
            accq = zeros
            for ch in range(16):
                g1 = plsc.load_gather(fdv, [idx1 + ch])
                g2 = plsc.load_gather(fdv, [idx2 + ch])
                dv = g1 - g2
                accq = accq + dv * dv
            hs[pl.ds(p * 16, 16)] = jnp.where(xm & ym, accq, zeros)

        def p_pair(q, _):
            p_store(q * 2)
            p_store(q * 2 + 1)
            return 0

        lax.fori_loop(0, 7, p_pair, 0)
        p_store(14)
        tvec = zeros
        for p in range(15):
            tvec = tvec + _vsqrt(hs[pl.ds(p * 16, 16)])
        total = hsum_full(tvec)
        xr_all = plsc.load_gather(pp, [lanes * 16 + xlane])
        cx = hsum_full(jnp.where(xr_all >= 0, _F32(1.0), _F32(0.0)))
        cy = hsum_full(jnp.where(ym, _F32(1.0), _F32(0.0)))
        outv[...] = total / (cx * cy)
        pltpu.sync_copy(outv, shd.at[pl.ds(sid * 16, 16)])

    plsc.subcore_barrier()

    @pl.when((cid == 0) & (sid == 0))
    def _():
        pltpu.sync_copy(shd, hs.at[pl.ds(0, 64)])
        dp0 = hs[pl.ds(0, 16)]
        dp1 = hs[pl.ds(16, 16)]
        dn0 = hs[pl.ds(32, 16)]
        dn1 = hs[pl.ds(48, 16)]
        loss = (jnp.maximum(dp0 - dn0 + _F32(0.1), zeros)
                + jnp.maximum(dp1 - dn1 + _F32(0.1), zeros))
        outv[...] = loss
        pltpu.sync_copy(outv.at[pl.ds(0, 1)], out_hbm)


@jax.jit
def _run(fd_flat):
    mesh = plsc.VectorSubcoreMesh(core_axis_name="c", subcore_axis_name="s",
                                  num_cores=1)
    k = functools.partial(
        pl.kernel,
        mesh=mesh,
        out_type=jax.ShapeDtypeStruct((1,), _F32),
        compiler_params=pltpu.CompilerParams(needs_layout_passes=False),
        scratch_types=[
            pltpu.VMEM((6144,), _F32),
            pltpu.VMEM((1024,), _F32),
            pltpu.VMEM((1024,), _F32),
            pltpu.VMEM((1024,), _F32),
            pltpu.VMEM((1024,), _F32),
            pltpu.VMEM((1024,), _F32),
            pltpu.VMEM((512,), _I32),
            pltpu.VMEM((256,), _F32),
            pltpu.VMEM((1024,), _F32),
            pltpu.VMEM((16,), _F32),
            pltpu.VMEM_SHARED((512,), _F32),
            pltpu.VMEM_SHARED((512,), _I32),
            pltpu.VMEM_SHARED((64,), _F32),
            pltpu.SemaphoreType.DMA,
        ],
    )(_sc_body)
    return k(fd_flat)


def kernel(feature_data):
    fd_flat = jnp.asarray(feature_data, dtype=_F32).reshape(6144)
    return _run(fd_flat)


# speedup vs baseline: 1.3339x; 1.0106x over previous
import functools

import jax
import jax.numpy as jnp
from jax import lax
from jax.experimental import pallas as pl
from jax.experimental.pallas import tpu as pltpu
from jax.experimental.pallas import tpu_sc as plsc

_F32 = jnp.float32
_I32 = jnp.int32


def _vsqrt(x):
    xi = lax.bitcast_convert_type(x, _I32)
    yi = jnp.int32(0x5F3759DF) - (xi >> 1)
    y = lax.bitcast_convert_type(yi, _F32)
    for _ in range(4):
        y = y * (1.5 - 0.5 * x * y * y)
    return x * y


def _splat_i32(v):
    return jnp.zeros((16,), _I32) + v


def _sc_body(fd_hbm, out_hbm, fdv, d0, d1, d2, d3, rr, pp, hs, mcol,
             outv, shm, shp, shd, sem):
    cid = lax.axis_index("c")
    sid = lax.axis_index("s")
    lanes = lax.iota(_I32, 16)
    zeros = jnp.zeros((16,), _F32)

    def hsum_full(x):
        for s in (8, 4, 2, 1):
            hs[pl.ds(240, 16)] = x
            x = x + plsc.load_gather(hs, [240 + (lanes ^ s)])
        return x

    @pl.when((cid == 0) & (sid < 8))
    def _():
        p = sid
        dirx = p < 4
        tb = p % 4
        t = 1 + tb // 2
        b = tb % 2
        rowmul = jnp.where(dirx, 128, 16)
        wmul = jnp.where(dirx, 16, 128)
        scale = jnp.where(dirx, _F32(0.125), _F32(1.0))
        base1 = b * 1024
        base2 = (t * 2 + b) * 1024
        h1 = pltpu.async_copy(fd_hbm.at[pl.ds(base1, 1024)],
                              fdv.at[pl.ds(base1, 1024)], sem)
        h2 = pltpu.async_copy(fd_hbm.at[pl.ds(base2, 1024)],
                              fdv.at[pl.ds(base2, 1024)], sem)
        h1.wait()
        h2.wait()

        colb = [base2 + j * rowmul for j in range(8)]
        woff = [w * wmul for w in range(8)]

        def m_group(g, _):
            ga = base1 + (g * 2) * rowmul
            oas = (ga, ga + rowmul)
            for k in range(16):
                oa = oas[k // 8]
                ob = colb[k % 8]
                acc = zeros
                for w in range(8):
                    av = fdv[pl.ds(oa + woff[w], 16)]
                    bv = fdv[pl.ds(ob + woff[w], 16)]
                    dv = av - bv
                    acc = acc + dv * dv
                hs[pl.ds(k * 16, 16)] = acc
            totals = zeros
            for ch in range(16):
                totals = totals + plsc.load_gather(hs, [lanes * 16 + ch])
            mvec = _vsqrt(totals * scale) + _F32(1e-08)
            mcol[pl.ds(g * 16, 16)] = mvec
            return 0

        lax.fori_loop(0, 4, m_group, 0)
        pltpu.sync_copy(mcol.at[pl.ds(0, 64)], shm.at[pl.ds(p * 64, 64)])

    plsc.subcore_barrier()

    @pl.when((cid == 0) & (sid < 4))
    def _():
        pltpu.sync_copy(shm, mcol.at[pl.ds(0, 512)])
        l64 = lanes * 64

        def mrow(c):
            return plsc.load_gather(mcol, [l64 + c])

        m0 = mrow(0)
        d0[pl.ds(0, 16)] = m0
        d1[pl.ds(0, 16)] = zeros - 1.0
        d2[pl.ds(0, 16)] = zeros - 1.0
        d3[pl.ds(0, 16)] = zeros + 1.0
        rr[pl.ds(0, 16)] = m0

        def i_edge(i, _):
            c = i * 8
            v0 = mrow(c) + d0[pl.ds((c - 8) * 16, 16)]
            v3 = d3[pl.ds((c - 8) * 16, 16)] + 1.0
            d0[pl.ds(c * 16, 16)] = v0
            d1[pl.ds(c * 16, 16)] = zeros + (i - 1).astype(_F32)
            d2[pl.ds(c * 16, 16)] = zeros
            d3[pl.ds(c * 16, 16)] = v3
            rr[pl.ds(c * 16, 16)] = v0 / v3
            return 0

        lax.fori_loop(1, 8, i_edge, 0)

        def j_edge(j, _):
            v0 = mrow(j) + d0[pl.ds((j - 1) * 16, 16)]
            v3 = d3[pl.ds((j - 1) * 16, 16)] + 1.0
            d0[pl.ds(j * 16, 16)] = v0
            d1[pl.ds(j * 16, 16)] = zeros
            d2[pl.ds(j * 16, 16)] = zeros + (j - 1).astype(_F32)
            d3[pl.ds(j * 16, 16)] = v3
            rr[pl.ds(j * 16, 16)] = v0 / v3
            return 0

        lax.fori_loop(1, 8, j_edge, 0)

        def dp_row(i, _):
            base = i * 128
            fi = zeros + i.astype(_F32)
            lcar = (d0[pl.ds(base, 16)], d3[pl.ds(base, 16)],
                    rr[pl.ds(base, 16)])

            def dp_cell(j, lcar):
                d0l, d3l, c2 = lcar
                c16 = base + j * 16
                c1 = rr[pl.ds(c16 - 128, 16)]
                c3 = rr[pl.ds(c16 - 144, 16)]
                b1 = (c1 <= c2) & (c1 <= c3)
                b2 = c2 <= c3
                d0u = d0[pl.ds(c16 - 128, 16)]
                d0g = d0[pl.ds(c16 - 144, 16)]
                d3u = d3[pl.ds(c16 - 128, 16)]
                d3g = d3[pl.ds(c16 - 144, 16)]
                v0 = plsc.load_gather(mcol, [l64 + (c16 >> 4)]) + jnp.where(
                    b1, d0u, jnp.where(b2, d0l, d0g))
                v3 = 1.0 + jnp.where(b1, d3u, jnp.where(b2, d3l, d3g))
                fj = zeros + j.astype(_F32)
                vr = v0 / v3
                d0[pl.ds(c16, 16)] = v0
                d1[pl.ds(c16, 16)] = jnp.where((~b1) & b2, fi, fi - 1.0)
                d2[pl.ds(c16, 16)] = jnp.where(b1, fj, fj - 1.0)
                d3[pl.ds(c16, 16)] = v3
                rr[pl.ds(c16, 16)] = vr
                return (v0, v3, vr)

            lax.fori_loop(1, 8, dp_cell, lcar)
            return 0

        lax.fori_loop(1, 8, dp_row, 0)

        def bt_step(tt, carry):
            ii, jj = carry
            pp[pl.ds(tt * 16, 16)] = ii
            pp[pl.ds(256 + tt * 16, 16)] = jj
            valid = ii >= 0
            ci = jnp.clip(ii, 0, 7)
            cj = jnp.clip(jj, 0, 7)
            idx = (ci * 8 + cj) * 16 + lanes
            n1 = plsc.load_gather(d1, [idx]).astype(_I32)
            n2 = plsc.load_gather(d2, [idx]).astype(_I32)
            return (jnp.where(valid, n1, ii), jnp.where(valid, n2, jj))

        seven = _splat_i32(7)
        lax.fori_loop(0, 16, bt_step, (seven, seven))

    @pl.when((cid == 0) & (sid < 4))
    def _():
        b = sid & 1
        t = 1 + (sid >> 1)
        xlane = (t - 1) * 2 + b
        ylane = 4 + xlane
        yj_raw = plsc.load_gather(pp, [lanes * 16 + ylane])
        yb_raw = plsc.load_gather(pp, [256 + lanes * 16 + ylane])
        ym = yj_raw >= 0
        jdx1 = b * 1024 + jnp.clip(yj_raw, 0, 7) * 16
        jdx2 = (t * 2 + b) * 1024 + jnp.clip(yb_raw, 0, 7) * 16

        def p_store(p):
            xr = plsc.load_gather(pp, [_splat_i32(p * 16 + xlane)])
            ar = plsc.load_gather(pp, [_splat_i32(256 + p * 16 + xlane)])
            xm = xr >= 0
            idx1 = jdx1 + jnp.clip(xr, 0, 7) * 128
            idx2 = jdx2 + jnp.clip(ar, 0, 7) * 128
            accq = zeros
            for ch in range(16):
                g1 = plsc.load_gather(fdv, [idx1 + ch])
                g2 = plsc.load_gather(fdv, [idx2 + ch])
                dv = g1 - g2
                accq = accq + dv * dv
            hs[pl.ds(p * 16, 16)] = jnp.where(xm & ym, accq, zeros)

        def p_pair(q, _):
            p_store(q * 2)
            p_store(q * 2 + 1)
            return 0

        lax.fori_loop(0, 7, p_pair, 0)
        p_store(14)
        tvec = zeros
        for p in range(15):
            tvec = tvec + _vsqrt(hs[pl.ds(p * 16, 16)])
        total = hsum_full(tvec)
        xr_all = plsc.load_gather(pp, [lanes * 16 + xlane])
        cx = hsum_full(jnp.where(xr_all >= 0, _F32(1.0), _F32(0.0)))
        cy = hsum_full(jnp.where(ym, _F32(1.0), _F32(0.0)))
        outv[...] = total / (cx * cy)
        pltpu.sync_copy(outv, shd.at[pl.ds(sid * 16, 16)])

    plsc.subcore_barrier()

    @pl.when((cid == 0) & (sid == 0))
    def _():
        pltpu.sync_copy(shd, hs.at[pl.ds(0, 64)])
        dp0 = hs[pl.ds(0, 16)]
        dp1 = hs[pl.ds(16, 16)]
        dn0 = hs[pl.ds(32, 16)]
        dn1 = hs[pl.ds(48, 16)]
        loss = (jnp.maximum(dp0 - dn0 + _F32(0.1), zeros)
                + jnp.maximum(dp1 - dn1 + _F32(0.1), zeros))
        outv[...] = loss
        pltpu.sync_copy(outv.at[pl.ds(0, 1)], out_hbm)


@jax.jit
def _run(fd_flat):
    mesh = plsc.VectorSubcoreMesh(core_axis_name="c", subcore_axis_name="s",
                                  num_cores=1)
    k = functools.partial(
        pl.kernel,
        mesh=mesh,
        out_type=jax.ShapeDtypeStruct((1,), _F32),
        compiler_params=pltpu.CompilerParams(needs_layout_passes=False),
        scratch_types=[
            pltpu.VMEM((6144,), _F32),
            pltpu.VMEM((1024,), _F32),
            pltpu.VMEM((1024,), _F32),
            pltpu.VMEM((1024,), _F32),
            pltpu.VMEM((1024,), _F32),
            pltpu.VMEM((1024,), _F32),
            pltpu.VMEM((512,), _I32),
            pltpu.VMEM((256,), _F32),
            pltpu.VMEM((1024,), _F32),
            pltpu.VMEM((16,), _F32),
            pltpu.VMEM_SHARED((512,), _F32),
            pltpu.VMEM_SHARED((512,), _I32),
            pltpu.VMEM_SHARED((64,), _F32),
            pltpu.SemaphoreType.DMA,
        ],
    )(_sc_body)
    return k(fd_flat)


def kernel(feature_data):
    fd_flat = jnp.asarray(feature_data, dtype=_F32).reshape(6144)
    return _run(fd_flat)
